# baseline (speedup 1.0000x reference)
import jax
import jax.numpy as jnp
from jax.experimental import pallas as pl
from jax.experimental.pallas import tpu as pltpu

N = 10000
BS = 50
DF = 128
NF = 8
D0 = 256

BLK = 2000
NBLK = N // BLK


def _dot(a, b, trans_lhs=False):
    dims = (((0,), (0,)) if trans_lhs else ((1,), (0,)), ((), ()))
    return jax.lax.dot_general(a, b, dims, preferred_element_type=jnp.float32)


def _onehot(batch_blk):
    seg_ids = jax.lax.broadcasted_iota(jnp.int32, (BLK, BS), 1)
    return (batch_blk == seg_ids).astype(jnp.float32)


def _pair_fold(w_pers):
    j = jax.lax.broadcasted_iota(jnp.int32, (NF, 2 * NF), 0)
    k = jax.lax.broadcasted_iota(jnp.int32, (NF, 2 * NF), 1)
    sel = ((k == 2 * j) | (k == 2 * j + 1)).astype(jnp.float32)
    return _dot(sel, w_pers)


def _blk(ref, i):
    return ref.at[pl.ds(i * BLK, BLK), :]


def _mono_kernel(x_hbm, b_ref, w1_ref, b1_ref, w2_ref, b2_ref,
                 g1_ref, g1b_ref, l1_ref, g2_ref, g2b_ref, l2_ref,
                 bng_ref, bnb_ref, out_hbm,
                 x_s, fv_s, h_s, h2_s, o_s, xsem, osem):
    xcp = [pltpu.make_async_copy(_blk(x_hbm, i), _blk(x_s, i), xsem.at[i])
           for i in range(NBLK)]
    for c in xcp:
        c.start()

    w1 = w1_ref[...]
    w2 = w2_ref[...]
    b1 = b1_ref[...]
    b2 = b2_ref[...]

    segx = jnp.zeros((BS, DF), jnp.float32)
    segf = jnp.zeros((BS, NF), jnp.float32)
    cnt = jnp.zeros((1, BS), jnp.float32)
    for i in range(NBLK):
        xcp[i].wait()
        xv = x_s[pl.ds(i * BLK, BLK), :]
        p1 = jnp.maximum(_dot(xv, w1) + b1, 0.0)
        fv = _dot(p1, w2) + b2
        fv_s[pl.ds(i * BLK, BLK), :] = fv
        oh = _onehot(b_ref[pl.ds(i * BLK, BLK), :])
        segx += _dot(oh, xv, trans_lhs=True)
        segf += _dot(oh, fv, trans_lhs=True)
        cnt += jnp.sum(oh, axis=0, keepdims=True)
    inv = 1.0 / jnp.maximum(cnt, 1.0)

    g1x = g1_ref[0:DF, :]
    g1p = _pair_fold(g1_ref[DF:DF + 2 * NF, :])
    l1x = l1_ref[0:DF, :]
    l1p = _pair_fold(l1_ref[DF:DF + 2 * NF, :])

    m1 = _dot(segx, l1x) + _dot(segf, l1p)
    g1b = g1b_ref[...]
    segh = jnp.zeros((BS, D0), jnp.float32)
    for i in range(NBLK):
        xv = x_s[pl.ds(i * BLK, BLK), :]
        fv = fv_s[pl.ds(i * BLK, BLK), :]
        oh = _onehot(b_ref[pl.ds(i * BLK, BLK), :])
        g = _dot(xv, g1x) + _dot(fv, g1p) + g1b
        h = jnp.maximum(g - _dot(oh * inv, m1), 0.0)
        h_s[pl.ds(i * BLK, BLK), :] = h
        segh += _dot(oh, h, trans_lhs=True)

    m2 = _dot(segh, l2_ref[...])
    g2 = g2_ref[...]
    g2b = g2b_ref[...]
    s1 = jnp.zeros((1, DF), jnp.float32)
    s2 = jnp.zeros((1, DF), jnp.float32)
    for i in range(NBLK):
        h = h_s[pl.ds(i * BLK, BLK), :]
        oh = _onehot(b_ref[pl.ds(i * BLK, BLK), :])
        h2 = _dot(h, g2) + g2b - _dot(oh * inv, m2)
        h2_s[pl.ds(i * BLK, BLK), :] = h2
        s1 += jnp.sum(h2, axis=0, keepdims=True)
        s2 += jnp.sum(h2 * h2, axis=0, keepdims=True)

    mu = s1 * (1.0 / N)
    var = s2 * (1.0 / N) - mu * mu
    scale = jax.lax.rsqrt(var + 1e-5) * bng_ref[...]
    bnb = bnb_ref[...]
    ocp = [pltpu.make_async_copy(_blk(o_s, i), _blk(out_hbm, i), osem.at[i])
           for i in range(NBLK)]
    for i in range(NBLK):
        xv = x_s[pl.ds(i * BLK, BLK), :]
        h2 = h2_s[pl.ds(i * BLK, BLK), :]
        o_s[pl.ds(i * BLK, BLK), :] = xv + (h2 - mu) * scale + bnb
        ocp[i].start()
    for c in ocp:
        c.wait()


def kernel(x, edge_index, batch, vertex_slices, edge_slices, rand_u,
           W1, b1, W2, b2, G1_W, G1_b, L1_W, G2_W, G2_b, L2_W, bn_g, bn_b):
    row = lambda v: v.reshape(1, -1)
    b2d = batch.reshape(N, 1)
    f32 = jnp.float32
    vmem = pl.BlockSpec(memory_space=pltpu.VMEM)
    out = pl.pallas_call(
        _mono_kernel,
        out_shape=jax.ShapeDtypeStruct((N, DF), f32),
        in_specs=[pl.BlockSpec(memory_space=pltpu.MemorySpace.HBM)] + [vmem] * 13,
        out_specs=pl.BlockSpec(memory_space=pltpu.MemorySpace.HBM),
        scratch_shapes=[pltpu.VMEM((N, DF), f32),
                        pltpu.VMEM((N, NF), f32),
                        pltpu.VMEM((N, D0), f32),
                        pltpu.VMEM((N, DF), f32),
                        pltpu.VMEM((N, DF), f32),
                        pltpu.SemaphoreType.DMA((NBLK,)),
                        pltpu.SemaphoreType.DMA((NBLK,))],
    )(x, b2d, W1, row(b1), W2, row(b2),
      G1_W, row(G1_b), L1_W, G2_W, row(G2_b), L2_W, row(bn_g), row(bn_b))
    return out


# speedup vs baseline: 1.0118x; 1.0118x over previous
import jax
import jax.numpy as jnp
from jax.experimental import pallas as pl
from jax.experimental.pallas import tpu as pltpu

N = 10000
BS = 50
DF = 128
NF = 8
D0 = 256

BLK = 2000
NBLK = N // BLK


def _dot(a, b, trans_lhs=False):
    dims = (((0,), (0,)) if trans_lhs else ((1,), (0,)), ((), ()))
    return jax.lax.dot_general(a, b, dims, preferred_element_type=jnp.float32)


def _onehot(batch_blk):
    seg_ids = jax.lax.broadcasted_iota(jnp.int32, (BLK, BS), 1)
    return (batch_blk == seg_ids).astype(jnp.float32)


def _pair_fold(w_pers):
    j = jax.lax.broadcasted_iota(jnp.int32, (NF, 2 * NF), 0)
    k = jax.lax.broadcasted_iota(jnp.int32, (NF, 2 * NF), 1)
    sel = ((k == 2 * j) | (k == 2 * j + 1)).astype(jnp.float32)
    return _dot(sel, w_pers)


def _mono_kernel(x_ref, b_ref, w1_ref, b1_ref, w2_ref, b2_ref,
                 g1_ref, g1b_ref, l1_ref, g2_ref, g2b_ref, l2_ref,
                 bng_ref, bnb_ref, out_ref, fv_s, h_s, h2_s):
    w1 = w1_ref[...]
    w2 = w2_ref[...]
    b1 = b1_ref[...]
    b2 = b2_ref[...]

    segx = jnp.zeros((BS, DF), jnp.float32)
    segf = jnp.zeros((BS, NF), jnp.float32)
    cnt = jnp.zeros((1, BS), jnp.float32)
    for i in range(NBLK):
        xv = x_ref[pl.ds(i * BLK, BLK), :]
        p1 = jnp.maximum(_dot(xv, w1) + b1, 0.0)
        fv = _dot(p1, w2) + b2
        fv_s[pl.ds(i * BLK, BLK), :] = fv
        oh = _onehot(b_ref[pl.ds(i * BLK, BLK), :])
        segx += _dot(oh, xv, trans_lhs=True)
        segf += _dot(oh, fv, trans_lhs=True)
        cnt += jnp.sum(oh, axis=0, keepdims=True)
    inv = 1.0 / jnp.maximum(cnt, 1.0)

    g1x = g1_ref[0:DF, :]
    g1p = _pair_fold(g1_ref[DF:DF + 2 * NF, :])
    l1x = l1_ref[0:DF, :]
    l1p = _pair_fold(l1_ref[DF:DF + 2 * NF, :])

    m1 = _dot(segx, l1x) + _dot(segf, l1p)
    g1b = g1b_ref[...]
    segh = jnp.zeros((BS, D0), jnp.float32)
    for i in range(NBLK):
        xv = x_ref[pl.ds(i * BLK, BLK), :]
        fv = fv_s[pl.ds(i * BLK, BLK), :]
        oh = _onehot(b_ref[pl.ds(i * BLK, BLK), :])
        g = _dot(xv, g1x) + _dot(fv, g1p) + g1b
        h = jnp.maximum(g - _dot(oh * inv, m1), 0.0)
        h_s[pl.ds(i * BLK, BLK), :] = h
        segh += _dot(oh, h, trans_lhs=True)

    m2 = _dot(segh, l2_ref[...])
    g2 = g2_ref[...]
    g2b = g2b_ref[...]
    s1 = jnp.zeros((1, DF), jnp.float32)
    s2 = jnp.zeros((1, DF), jnp.float32)
    for i in range(NBLK):
        h = h_s[pl.ds(i * BLK, BLK), :]
        oh = _onehot(b_ref[pl.ds(i * BLK, BLK), :])
        h2 = _dot(h, g2) + g2b - _dot(oh * inv, m2)
        h2_s[pl.ds(i * BLK, BLK), :] = h2
        s1 += jnp.sum(h2, axis=0, keepdims=True)
        s2 += jnp.sum(h2 * h2, axis=0, keepdims=True)

    mu = s1 * (1.0 / N)
    var = s2 * (1.0 / N) - mu * mu
    scale = jax.lax.rsqrt(var + 1e-5) * bng_ref[...]
    bnb = bnb_ref[...]
    for i in range(NBLK):
        xv = x_ref[pl.ds(i * BLK, BLK), :]
        h2 = h2_s[pl.ds(i * BLK, BLK), :]
        out_ref[pl.ds(i * BLK, BLK), :] = xv + (h2 - mu) * scale + bnb


def kernel(x, edge_index, batch, vertex_slices, edge_slices, rand_u,
           W1, b1, W2, b2, G1_W, G1_b, L1_W, G2_W, G2_b, L2_W, bn_g, bn_b):
    row = lambda v: v.reshape(1, -1)
    b2d = batch.reshape(N, 1)
    f32 = jnp.float32
    out = pl.pallas_call(
        _mono_kernel,
        out_shape=jax.ShapeDtypeStruct((N, DF), f32),
        scratch_shapes=[pltpu.VMEM((N, NF), f32),
                        pltpu.VMEM((N, D0), f32),
                        pltpu.VMEM((N, DF), f32)],
    )(x, b2d, W1, row(b1), W2, row(b2),
      G1_W, row(G1_b), L1_W, G2_W, row(G2_b), L2_W, row(bn_g), row(bn_b))
    return out
